# Initial kernel scaffold; baseline (speedup 1.0000x reference)
import jax
import jax.numpy as jnp
from jax.experimental import pallas as pl


def kernel(node_feature, node_type, edge_time, edge_index, edge_type, adapt_W, adapt_b, Wk, bk, Wq, bq, Wv, bv, Wa, ba, rel_pri, rel_att, rel_msg, skip, rte_W, rte_b, cls_W, cls_b):
    raise NotImplementedError("write your pallas kernel here")


# speedup vs baseline: 14.4449x; 14.4449x over previous
import functools
import math

import numpy as np

import jax
import jax.numpy as jnp
from jax import lax
from jax.experimental import pallas as pl
from jax.experimental.pallas import tpu as pltpu
from jax.experimental.pallas import tpu_sc as plsc

N_NODES = 10000
N_EDGES = 320000
IN_DIM = 128
HID = 128
N_TYPES = 4
N_REL = 8
N_HEADS = 8
D_K = 16
N_LAYERS = 2
N_OUT = 16

NPAD = 10240
NB = 256
EB = 512

NW = 32
EPW = N_EDGES // NW
CHUNK = 80
NCHUNK = EPW // CHUNK
HALF = NPAD // 2
ACC_ROWS = HALF + 128
ZROWS = ACC_ROWS // 16
DROWS = HALF // 16
EPS = N_EDGES // 16
NCHUNK_S = EPS // CHUNK

_HI = jax.lax.Precision.HIGHEST


def _pe_table():
    t = np.arange(128, dtype=np.float64)[:, None]
    div = np.exp(np.arange(0, HID, 2, dtype=np.float64) * (-(math.log(10000.0) / HID)))
    ang = t * div[None, :]
    pe = np.stack([np.sin(ang), np.cos(ang)], axis=-1).reshape(128, HID)
    return jnp.asarray(pe / math.sqrt(HID), jnp.float32)


def _blockdiag(a):
    out = jnp.zeros(a.shape[:-3] + (HID, HID), jnp.float32)
    for h in range(N_HEADS):
        out = out.at[..., h * D_K:(h + 1) * D_K, h * D_K:(h + 1) * D_K].set(a[..., h, :, :])
    return out


def _adapt_body(nf, oh, W, b, out):
    x = nf[...]
    ohv = oh[...]
    acc = jnp.zeros((NB, HID), jnp.float32)
    for t in range(N_TYPES):
        y = jnp.tanh(jnp.dot(x, W[t], preferred_element_type=jnp.float32, precision=_HI)
                     + b[t][None, :])
        acc = acc + ohv[:, t][:, None] * y
    out[...] = acc


def _adapt(nf_p, oh_p, adapt_W, adapt_b):
    return pl.pallas_call(
        _adapt_body,
        grid=(NPAD // NB,),
        in_specs=[
            pl.BlockSpec((NB, IN_DIM), lambda i: (i, 0)),
            pl.BlockSpec((NB, N_TYPES), lambda i: (i, 0)),
            pl.BlockSpec((N_TYPES, IN_DIM, HID), lambda i: (0, 0, 0)),
            pl.BlockSpec((N_TYPES, HID), lambda i: (0, 0)),
        ],
        out_specs=pl.BlockSpec((NB, HID), lambda i: (i, 0)),
        out_shape=jax.ShapeDtypeStruct((NPAD, HID), jnp.float32),
    )(nf_p, oh_p, adapt_W, adapt_b)


def _tables_body(pe, rw, rb, Wk, Wv, Amsg, rk_out, rv_out):
    rte = jnp.dot(pe[...], rw[...], preferred_element_type=jnp.float32, precision=_HI) + rb[0][None, :]
    for t in range(N_TYPES):
        rk_out[t] = jnp.dot(rte, Wk[t], preferred_element_type=jnp.float32, precision=_HI)
        rv = jnp.dot(rte, Wv[t], preferred_element_type=jnp.float32, precision=_HI)
        for r in range(N_REL):
            rv_out[t, :, r, :] = jnp.dot(rv, Amsg[r], preferred_element_type=jnp.float32,
                                         precision=_HI)


def _tables(pe, rw, rb2, Wk_l, Wv_l, Amsg):
    rk, rv = pl.pallas_call(
        _tables_body,
        grid=(1,),
        in_specs=[
            pl.BlockSpec((128, HID), lambda i: (0, 0)),
            pl.BlockSpec((HID, HID), lambda i: (0, 0)),
            pl.BlockSpec((1, HID), lambda i: (0, 0)),
            pl.BlockSpec((N_TYPES, HID, HID), lambda i: (0, 0, 0)),
            pl.BlockSpec((N_TYPES, HID, HID), lambda i: (0, 0, 0)),
            pl.BlockSpec((N_REL, HID, HID), lambda i: (0, 0, 0)),
        ],
        out_specs=[
            pl.BlockSpec((N_TYPES, 128, HID), lambda i: (0, 0, 0)),
            pl.BlockSpec((N_TYPES, 128, N_REL, HID), lambda i: (0, 0, 0, 0)),
        ],
        out_shape=[
            jax.ShapeDtypeStruct((N_TYPES, 128, HID), jnp.float32),
            jax.ShapeDtypeStruct((N_TYPES, 128, N_REL, HID), jnp.float32),
        ],
    )(pe, rw, rb2, Wk_l, Wv_l, Amsg)
    return rk.reshape(N_TYPES * 128, HID), rv.reshape(N_TYPES * 128 * N_REL, HID)


def _nodepre_body(x, oh, Wk, bk, Wq, bq, Wv, bv, Aatt, Amsg, kn_out, qr_out, vr_out):
    xv = x[...]
    ohv = oh[...]

    def tlin(W, b):
        acc = jnp.zeros((NB, HID), jnp.float32)
        for t in range(N_TYPES):
            acc = acc + ohv[:, t][:, None] * (
                jnp.dot(xv, W[t], preferred_element_type=jnp.float32, precision=_HI)
                + b[t][None, :])
        return acc

    K = tlin(Wk, bk)
    Q = tlin(Wq, bq)
    V = tlin(Wv, bv)
    kn_out[...] = K
    for r in range(N_REL):
        qr_out[:, r, :] = jnp.dot(Q, Aatt[r], preferred_element_type=jnp.float32, precision=_HI)
        vr_out[:, r, :] = jnp.dot(V, Amsg[r], preferred_element_type=jnp.float32, precision=_HI)


def _nodepre(x, oh_p, Wk_l, bk_l, Wq_l, bq_l, Wv_l, bv_l, Aatt, Amsg):
    w3 = pl.BlockSpec((N_TYPES, HID, HID), lambda i: (0, 0, 0))
    b2 = pl.BlockSpec((N_TYPES, HID), lambda i: (0, 0))
    r3 = pl.BlockSpec((N_REL, HID, HID), lambda i: (0, 0, 0))
    kn, qr, vr = pl.pallas_call(
        _nodepre_body,
        grid=(NPAD // NB,),
        in_specs=[
            pl.BlockSpec((NB, HID), lambda i: (i, 0)),
            pl.BlockSpec((NB, N_TYPES), lambda i: (i, 0)),
            w3, b2, w3, b2, w3, b2, r3, r3,
        ],
        out_specs=[
            pl.BlockSpec((NB, HID), lambda i: (i, 0)),
            pl.BlockSpec((NB, N_REL, HID), lambda i: (i, 0, 0)),
            pl.BlockSpec((NB, N_REL, HID), lambda i: (i, 0, 0)),
        ],
        out_shape=[
            jax.ShapeDtypeStruct((NPAD, HID), jnp.float32),
            jax.ShapeDtypeStruct((NPAD, N_REL, HID), jnp.float32),
            jax.ShapeDtypeStruct((NPAD, N_REL, HID), jnp.float32),
        ],
    )(x, oh_p, Wk_l, bk_l, Wq_l, bq_l, Wv_l, bv_l, Aatt, Amsg)
    return kn, qr.reshape(NPAD * N_REL, HID), vr.reshape(NPAD * N_REL, HID)


def _att_body(kg, rkg, qg, vg, rvg, m_out, a_out):
    P = (kg[...] + rkg[...]) * qg[...]
    i0 = lax.broadcasted_iota(jnp.int32, (HID, 16), 0)
    j0 = lax.broadcasted_iota(jnp.int32, (HID, 16), 1)
    S16 = ((i0 // D_K) == j0).astype(jnp.float32)
    att16 = jnp.dot(P, S16, preferred_element_type=jnp.float32, precision=_HI)
    e16 = jnp.exp(jnp.minimum(att16, 80.0))
    i1 = lax.broadcasted_iota(jnp.int32, (16, HID), 0)
    j1 = lax.broadcasted_iota(jnp.int32, (16, HID), 1)
    Sm = (i1 == (j1 // D_K)).astype(jnp.float32)
    aexp = jnp.dot(e16, Sm, preferred_element_type=jnp.float32, precision=_HI)
    m_out[...] = aexp * (vg[...] + rvg[...])
    a_out[...] = aexp


def _att(kg, rkg, qg, vg, rvg):
    e2 = pl.BlockSpec((EB, HID), lambda i: (i, 0))
    return pl.pallas_call(
        _att_body,
        grid=(N_EDGES // EB,),
        in_specs=[e2, e2, e2, e2, e2],
        out_specs=[e2, e2],
        out_shape=[jax.ShapeDtypeStruct((N_EDGES, HID), jnp.float32),
                   jax.ShapeDtypeStruct((N_EDGES, HID), jnp.float32)],
    )(kg, rkg, qg, vg, rvg)


def _update_body(num_r, den_r, x, oh, Wa, ba, sig, out):
    denx = den_r[...] + 1e-16
    agg = num_r[...] / denx
    g = jax.nn.gelu(agg)
    ohv = oh[...]
    trans = jnp.zeros((NB, HID), jnp.float32)
    for t in range(N_TYPES):
        trans = trans + ohv[:, t][:, None] * (
            jnp.dot(g, Wa[t], preferred_element_type=jnp.float32, precision=_HI)
            + ba[t][None, :])
    alph = jnp.dot(ohv, sig[...], preferred_element_type=jnp.float32, precision=_HI)
    out[...] = trans * alph + x[...] * (1.0 - alph)


def _update(num, den, x, oh_p, Wa_l, ba_l, sig_col):
    return pl.pallas_call(
        _update_body,
        grid=(NPAD // NB,),
        in_specs=[
            pl.BlockSpec((NB, HID), lambda i: (i, 0)),
            pl.BlockSpec((NB, HID), lambda i: (i, 0)),
            pl.BlockSpec((NB, HID), lambda i: (i, 0)),
            pl.BlockSpec((NB, N_TYPES), lambda i: (i, 0)),
            pl.BlockSpec((N_TYPES, HID, HID), lambda i: (0, 0, 0)),
            pl.BlockSpec((N_TYPES, HID), lambda i: (0, 0)),
            pl.BlockSpec((N_TYPES, 1), lambda i: (0, 0)),
        ],
        out_specs=pl.BlockSpec((NB, HID), lambda i: (i, 0)),
        out_shape=jax.ShapeDtypeStruct((NPAD, HID), jnp.float32),
    )(num, den, x, oh_p, Wa_l, ba_l, sig_col)


def _cls_body(x, W, b, out):
    logits = jnp.dot(x[...], W[...], preferred_element_type=jnp.float32, precision=_HI) + b[0][None, :]
    m = jnp.max(logits, axis=-1, keepdims=True)
    z = logits - m
    out[...] = z - jnp.log(jnp.sum(jnp.exp(z), axis=-1, keepdims=True))


def _cls(x, cls_W, cls_b2):
    return pl.pallas_call(
        _cls_body,
        grid=(NPAD // NB,),
        in_specs=[
            pl.BlockSpec((NB, HID), lambda i: (i, 0)),
            pl.BlockSpec((HID, N_OUT), lambda i: (0, 0)),
            pl.BlockSpec((1, N_OUT), lambda i: (0, 0)),
        ],
        out_specs=pl.BlockSpec((NB, N_OUT), lambda i: (i, 0)),
        out_shape=jax.ShapeDtypeStruct((NPAD, N_OUT), jnp.float32),
    )(x, cls_W, cls_b2)


def _sc_gather_body(src_h, dst_h, et_h, tm_h, nt_h,
                    kn_h, rtek_h, qr_h, vr_h, rtev_h,
                    kg_h, rkg_h, qg_h, vg_h, rvg_h,
                    ntb, srcb, dstb, etb, tmb, ikb, iqb, ivb, irb,
                    bk, brk, bq, bv, brv, sem):
    cid = lax.axis_index("c")
    sid = lax.axis_index("s")
    wid = sid * 2 + cid
    pltpu.sync_copy(nt_h, ntb)

    def chunk(c, carry):
        base = wid * EPW + c * CHUNK
        pltpu.sync_copy(src_h.at[pl.ds(base, CHUNK)], srcb)
        pltpu.sync_copy(dst_h.at[pl.ds(base, CHUNK)], dstb)
        pltpu.sync_copy(et_h.at[pl.ds(base, CHUNK)], etb)
        pltpu.sync_copy(tm_h.at[pl.ds(base, CHUNK)], tmb)
        for j in range(CHUNK // 16):
            sl = pl.ds(j * 16, 16)
            s = srcb[sl]
            d = dstb[sl]
            e = etb[sl]
            t = tmb[sl]
            st = plsc.load_gather(ntb, [s])
            ik = st * 128 + t
            ikb[sl] = ik
            iqb[sl] = d * N_REL + e
            ivb[sl] = s * N_REL + e
            irb[sl] = ik * N_REL + e
        c1 = pltpu.async_copy(kn_h.at[srcb], bk, sem)
        c2 = pltpu.async_copy(rtek_h.at[ikb], brk, sem)
        c3 = pltpu.async_copy(qr_h.at[iqb], bq, sem)
        c4 = pltpu.async_copy(vr_h.at[ivb], bv, sem)
        c5 = pltpu.async_copy(rtev_h.at[irb], brv, sem)
        c1.wait()
        c2.wait()
        c3.wait()
        c4.wait()
        c5.wait()
        pltpu.sync_copy(bk, kg_h.at[pl.ds(base, CHUNK)])
        pltpu.sync_copy(brk, rkg_h.at[pl.ds(base, CHUNK)])
        pltpu.sync_copy(bq, qg_h.at[pl.ds(base, CHUNK)])
        pltpu.sync_copy(bv, vg_h.at[pl.ds(base, CHUNK)])
        pltpu.sync_copy(brv, rvg_h.at[pl.ds(base, CHUNK)])
        return carry

    lax.fori_loop(0, NCHUNK, chunk, 0)


def _sc_gather(src, dst, et, tm, nt, kn, rtek, qr, vr, rtev):
    mesh = plsc.VectorSubcoreMesh(core_axis_name="c", subcore_axis_name="s")
    eout = jax.ShapeDtypeStruct((N_EDGES, HID), jnp.float32)
    f = functools.partial(
        pl.kernel,
        out_type=(eout,) * 5,
        mesh=mesh,
        scratch_types=[
            pltpu.VMEM((N_NODES,), jnp.int32),
            pltpu.VMEM((CHUNK,), jnp.int32),
            pltpu.VMEM((CHUNK,), jnp.int32),
            pltpu.VMEM((CHUNK,), jnp.int32),
            pltpu.VMEM((CHUNK,), jnp.int32),
            pltpu.VMEM((CHUNK,), jnp.int32),
            pltpu.VMEM((CHUNK,), jnp.int32),
            pltpu.VMEM((CHUNK,), jnp.int32),
            pltpu.VMEM((CHUNK,), jnp.int32),
            pltpu.VMEM((CHUNK, HID), jnp.float32),
            pltpu.VMEM((CHUNK, HID), jnp.float32),
            pltpu.VMEM((CHUNK, HID), jnp.float32),
            pltpu.VMEM((CHUNK, HID), jnp.float32),
            pltpu.VMEM((CHUNK, HID), jnp.float32),
            pltpu.SemaphoreType.DMA,
        ],
        compiler_params=pltpu.CompilerParams(needs_layout_passes=False),
    )(_sc_gather_body)
    return f(src, dst, et, tm, nt, kn, rtek, qr, vr, rtev)


def _sc_scatter_body(m_h, a_h, dst_h, num_h, den_h, shared, tmp, mab, dstb, idxb):
    cid = lax.axis_index("c")
    sid = lax.axis_index("s")
    off = cid * HALF
    zrows = pl.ds(sid * ZROWS, ZROWS)
    drows = pl.ds(sid * DROWS, DROWS)

    def zero_tmp():
        def zrow(i, carry):
            for j in range(HID // 16):
                tmp[i, pl.ds(j * 16, 16)] = jnp.zeros((16,), jnp.float32)
            return carry
        lax.fori_loop(0, ZROWS, zrow, 0)

    def scatter_pass(src_h):
        def chunk(c, carry):
            base = sid * EPS + c * CHUNK
            pltpu.sync_copy(dst_h.at[pl.ds(base, CHUNK)], dstb)
            pltpu.sync_copy(src_h.at[pl.ds(base, CHUNK)], mab)
            for j in range(CHUNK // 16):
                sl = pl.ds(j * 16, 16)
                local = dstb[sl] - off
                ok = (local >= 0) & (local < HALF)
                idxb[sl] = jnp.where(ok, local, HALF)
            pltpu.sync_copy(mab, shared.at[idxb], add=True)
            return carry
        lax.fori_loop(0, NCHUNK_S, chunk, 0)

    def one_kind(src_h, out_h):
        zero_tmp()
        pltpu.sync_copy(tmp.at[pl.ds(0, ZROWS)], shared.at[zrows])
        plsc.subcore_barrier()
        scatter_pass(src_h)
        plsc.subcore_barrier()
        pltpu.sync_copy(shared.at[pl.ds(sid * DROWS, DROWS)], tmp.at[pl.ds(0, DROWS)])
        pltpu.sync_copy(tmp.at[pl.ds(0, DROWS)], out_h.at[pl.ds(off + sid * DROWS, DROWS)])

    one_kind(m_h, num_h)
    one_kind(a_h, den_h)


def _sc_scatter(m, a, dst):
    mesh = plsc.VectorSubcoreMesh(core_axis_name="c", subcore_axis_name="s")
    f = functools.partial(
        pl.kernel,
        out_type=(jax.ShapeDtypeStruct((NPAD, HID), jnp.float32),
                  jax.ShapeDtypeStruct((NPAD, HID), jnp.float32)),
        mesh=mesh,
        scratch_types=[
            pltpu.VMEM_SHARED((ACC_ROWS, HID), jnp.float32),
            pltpu.VMEM((ZROWS, HID), jnp.float32),
            pltpu.VMEM((CHUNK, HID), jnp.float32),
            pltpu.VMEM((CHUNK,), jnp.int32),
            pltpu.VMEM((CHUNK,), jnp.int32),
        ],
        compiler_params=pltpu.CompilerParams(needs_layout_passes=False),
    )(_sc_scatter_body)
    return f(m, a, dst)


def kernel(node_feature, node_type, edge_time, edge_index, edge_type,
           adapt_W, adapt_b, Wk, bk, Wq, bq, Wv, bv, Wa, ba,
           rel_pri, rel_att, rel_msg, skip, rte_W, rte_b, cls_W, cls_b):
    nt = node_type.astype(jnp.int32)
    src = edge_index[0].astype(jnp.int32)
    dst = edge_index[1].astype(jnp.int32)
    et = edge_type.astype(jnp.int32)
    tm = edge_time.astype(jnp.int32)

    oh = (nt[:, None] == jnp.arange(N_TYPES, dtype=jnp.int32)[None, :]).astype(jnp.float32)
    oh_p = jnp.pad(oh, ((0, NPAD - N_NODES), (0, 0)))
    nf_p = jnp.pad(node_feature, ((0, NPAD - N_NODES), (0, 0)))

    pe = _pe_table()
    sig = jax.nn.sigmoid(skip)

    x = _adapt(nf_p, oh_p, adapt_W, adapt_b)

    for l in range(N_LAYERS):
        scale = jnp.repeat(rel_pri[l], D_K, axis=-1) / math.sqrt(D_K)
        Aatt = _blockdiag(jnp.swapaxes(rel_att[l], -1, -2)) * scale[:, None, :]
        Amsg = _blockdiag(rel_msg[l])

        rtek, rtev = _tables(pe, rte_W[l], rte_b[l][None, :], Wk[l], Wv[l], Amsg)
        kn, qr, vr = _nodepre(x, oh_p, Wk[l], bk[l], Wq[l], bq[l], Wv[l], bv[l], Aatt, Amsg)

        kg, rkg, qg, vg, rvg = _sc_gather(src, dst, et, tm, nt, kn, rtek, qr, vr, rtev)
        m, a = _att(kg, rkg, qg, vg, rvg)
        num, den = _sc_scatter(m, a, dst)
        x = _update(num, den, x, oh_p, Wa[l], ba[l], sig[l][:, None])

    out = _cls(x, cls_W, cls_b[None, :])
    return out[:N_NODES]


# speedup vs baseline: 15.4786x; 1.0716x over previous
import functools
import math

import numpy as np

import jax
import jax.numpy as jnp
from jax import lax
from jax.experimental import pallas as pl
from jax.experimental.pallas import tpu as pltpu
from jax.experimental.pallas import tpu_sc as plsc

N_NODES = 10000
N_EDGES = 320000
IN_DIM = 128
HID = 128
N_TYPES = 4
N_REL = 8
N_HEADS = 8
D_K = 16
N_LAYERS = 2
N_OUT = 16

NPAD = 10240
NB = 256
EPAD = N_EDGES + 128

NW = 32
EPW = N_EDGES // NW
CH = 16
NFULL = 625
DEN_W = N_NODES * N_HEADS

HALF = NPAD // 2
ACC_ROWS = HALF + 128
ZROWS = ACC_ROWS // 16
DROWS = HALF // 16
SCH = 80
EPS = N_EDGES // 16
NCHS = EPS // SCH

_HI = jax.lax.Precision.HIGHEST


def _pe_table():
    t = np.arange(128, dtype=np.float64)[:, None]
    div = np.exp(np.arange(0, HID, 2, dtype=np.float64) * (-(math.log(10000.0) / HID)))
    ang = t * div[None, :]
    pe = np.stack([np.sin(ang), np.cos(ang)], axis=-1).reshape(128, HID)
    return jnp.asarray(pe / math.sqrt(HID), jnp.float32)


def _blockdiag(a):
    out = jnp.zeros(a.shape[:-3] + (HID, HID), jnp.float32)
    for h in range(N_HEADS):
        out = out.at[..., h * D_K:(h + 1) * D_K, h * D_K:(h + 1) * D_K].set(a[..., h, :, :])
    return out


def _adapt_body(nf, oh, W, b, out):
    x = nf[...]
    ohv = oh[...]
    acc = jnp.zeros((NB, HID), jnp.float32)
    for t in range(N_TYPES):
        y = jnp.tanh(jnp.dot(x, W[t], preferred_element_type=jnp.float32, precision=_HI)
                     + b[t][None, :])
        acc = acc + ohv[:, t][:, None] * y
    out[...] = acc


def _adapt(nf_p, oh_p, adapt_W, adapt_b):
    return pl.pallas_call(
        _adapt_body,
        grid=(NPAD // NB,),
        in_specs=[
            pl.BlockSpec((NB, IN_DIM), lambda i: (i, 0)),
            pl.BlockSpec((NB, N_TYPES), lambda i: (i, 0)),
            pl.BlockSpec((N_TYPES, IN_DIM, HID), lambda i: (0, 0, 0)),
            pl.BlockSpec((N_TYPES, HID), lambda i: (0, 0)),
        ],
        out_specs=pl.BlockSpec((NB, HID), lambda i: (i, 0)),
        out_shape=jax.ShapeDtypeStruct((NPAD, HID), jnp.float32),
    )(nf_p, oh_p, adapt_W, adapt_b)


def _tables_body(pe, rw, rb, Wk, Wv, Amsg, rk_out, rv_out):
    rte = jnp.dot(pe[...], rw[...], preferred_element_type=jnp.float32, precision=_HI) + rb[0][None, :]
    for t in range(N_TYPES):
        rk_out[t] = jnp.dot(rte, Wk[t], preferred_element_type=jnp.float32, precision=_HI)
        rv = jnp.dot(rte, Wv[t], preferred_element_type=jnp.float32, precision=_HI)
        for r in range(N_REL):
            rv_out[t, :, r, :] = jnp.dot(rv, Amsg[r], preferred_element_type=jnp.float32,
                                         precision=_HI)


def _tables(pe, rw, rb2, Wk_l, Wv_l, Amsg):
    rk, rv = pl.pallas_call(
        _tables_body,
        grid=(1,),
        in_specs=[
            pl.BlockSpec((128, HID), lambda i: (0, 0)),
            pl.BlockSpec((HID, HID), lambda i: (0, 0)),
            pl.BlockSpec((1, HID), lambda i: (0, 0)),
            pl.BlockSpec((N_TYPES, HID, HID), lambda i: (0, 0, 0)),
            pl.BlockSpec((N_TYPES, HID, HID), lambda i: (0, 0, 0)),
            pl.BlockSpec((N_REL, HID, HID), lambda i: (0, 0, 0)),
        ],
        out_specs=[
            pl.BlockSpec((N_TYPES, 128, HID), lambda i: (0, 0, 0)),
            pl.BlockSpec((N_TYPES, 128, N_REL, HID), lambda i: (0, 0, 0, 0)),
        ],
        out_shape=[
            jax.ShapeDtypeStruct((N_TYPES, 128, HID), jnp.float32),
            jax.ShapeDtypeStruct((N_TYPES, 128, N_REL, HID), jnp.float32),
        ],
    )(pe, rw, rb2, Wk_l, Wv_l, Amsg)
    return rk.reshape(N_TYPES * 128, HID), rv.reshape(N_TYPES * 128 * N_REL, HID)


def _nodepre_body(x, oh, Wk, bk, Wq, bq, Wv, bv, Aatt, Amsg, kn_out, qr_out, vr_out):
    xv = x[...]
    ohv = oh[...]

    def tlin(W, b):
        acc = jnp.zeros((NB, HID), jnp.float32)
        for t in range(N_TYPES):
            acc = acc + ohv[:, t][:, None] * (
                jnp.dot(xv, W[t], preferred_element_type=jnp.float32, precision=_HI)
                + b[t][None, :])
        return acc

    K = tlin(Wk, bk)
    Q = tlin(Wq, bq)
    V = tlin(Wv, bv)
    kn_out[...] = K
    for r in range(N_REL):
        qr_out[:, r, :] = jnp.dot(Q, Aatt[r], preferred_element_type=jnp.float32, precision=_HI)
        vr_out[:, r, :] = jnp.dot(V, Amsg[r], preferred_element_type=jnp.float32, precision=_HI)


def _nodepre(x, oh_p, Wk_l, bk_l, Wq_l, bq_l, Wv_l, bv_l, Aatt, Amsg):
    w3 = pl.BlockSpec((N_TYPES, HID, HID), lambda i: (0, 0, 0))
    b2 = pl.BlockSpec((N_TYPES, HID), lambda i: (0, 0))
    r3 = pl.BlockSpec((N_REL, HID, HID), lambda i: (0, 0, 0))
    kn, qr, vr = pl.pallas_call(
        _nodepre_body,
        grid=(NPAD // NB,),
        in_specs=[
            pl.BlockSpec((NB, HID), lambda i: (i, 0)),
            pl.BlockSpec((NB, N_TYPES), lambda i: (i, 0)),
            w3, b2, w3, b2, w3, b2, r3, r3,
        ],
        out_specs=[
            pl.BlockSpec((NB, HID), lambda i: (i, 0)),
            pl.BlockSpec((NB, N_REL, HID), lambda i: (i, 0, 0)),
            pl.BlockSpec((NB, N_REL, HID), lambda i: (i, 0, 0)),
        ],
        out_shape=[
            jax.ShapeDtypeStruct((NPAD, HID), jnp.float32),
            jax.ShapeDtypeStruct((NPAD, N_REL, HID), jnp.float32),
            jax.ShapeDtypeStruct((NPAD, N_REL, HID), jnp.float32),
        ],
    )(x, oh_p, Wk_l, bk_l, Wq_l, bq_l, Wv_l, bv_l, Aatt, Amsg)
    return kn, qr.reshape(NPAD * N_REL, HID), vr.reshape(NPAD * N_REL, HID)


def _update_body(num_r, denp, x, oh, Wa, ba, sig, out):
    den8 = jnp.sum(denp[...], axis=0)
    i1 = lax.broadcasted_iota(jnp.int32, (N_HEADS, HID), 0)
    j1 = lax.broadcasted_iota(jnp.int32, (N_HEADS, HID), 1)
    Sm = (i1 == (j1 // D_K)).astype(jnp.float32)
    denx = jnp.dot(den8, Sm, preferred_element_type=jnp.float32, precision=_HI) + 1e-16
    agg = num_r[...] / denx
    g = jax.nn.gelu(agg)
    ohv = oh[...]
    trans = jnp.zeros((NB, HID), jnp.float32)
    for t in range(N_TYPES):
        trans = trans + ohv[:, t][:, None] * (
            jnp.dot(g, Wa[t], preferred_element_type=jnp.float32, precision=_HI)
            + ba[t][None, :])
    alph = jnp.dot(ohv, sig[...], preferred_element_type=jnp.float32, precision=_HI)
    out[...] = trans * alph + x[...] * (1.0 - alph)


def _update(num, denp, x, oh_p, Wa_l, ba_l, sig_col):
    return pl.pallas_call(
        _update_body,
        grid=(NPAD // NB,),
        in_specs=[
            pl.BlockSpec((NB, HID), lambda i: (i, 0)),
            pl.BlockSpec((NW, NB, N_HEADS), lambda i: (0, i, 0)),
            pl.BlockSpec((NB, HID), lambda i: (i, 0)),
            pl.BlockSpec((NB, N_TYPES), lambda i: (i, 0)),
            pl.BlockSpec((N_TYPES, HID, HID), lambda i: (0, 0, 0)),
            pl.BlockSpec((N_TYPES, HID), lambda i: (0, 0)),
            pl.BlockSpec((N_TYPES, 1), lambda i: (0, 0)),
        ],
        out_specs=pl.BlockSpec((NB, HID), lambda i: (i, 0)),
        out_shape=jax.ShapeDtypeStruct((NPAD, HID), jnp.float32),
    )(num, denp, x, oh_p, Wa_l, ba_l, sig_col)


def _cls_body(x, W, b, out):
    logits = jnp.dot(x[...], W[...], preferred_element_type=jnp.float32, precision=_HI) + b[0][None, :]
    m = jnp.max(logits, axis=-1, keepdims=True)
    z = logits - m
    out[...] = z - jnp.log(jnp.sum(jnp.exp(z), axis=-1, keepdims=True))


def _cls(x, cls_W, cls_b2):
    return pl.pallas_call(
        _cls_body,
        grid=(NPAD // NB,),
        in_specs=[
            pl.BlockSpec((NB, HID), lambda i: (i, 0)),
            pl.BlockSpec((HID, N_OUT), lambda i: (0, 0)),
            pl.BlockSpec((1, N_OUT), lambda i: (0, 0)),
        ],
        out_specs=pl.BlockSpec((NB, N_OUT), lambda i: (i, 0)),
        out_shape=jax.ShapeDtypeStruct((NPAD, N_OUT), jnp.float32),
    )(x, cls_W, cls_b2)


def _sc_edge_body(src_h, dst_h, ik_h, iq_h, iv_h, ir_h,
                  kn_h, rtek_h, qr_h, vr_h, rtev_h,
                  m_h, denp_h,
                  srcb0, dstb0, ikb0, iqb0, ivb0, irb0,
                  srcb1, dstb1, ikb1, iqb1, ivb1, irb1,
                  kb0, rkb0, qb0, vb0, rvb0,
                  kb1, rkb1, qb1, vb1, rvb1,
                  denf,
                  semI0, semI1, semG0, semG1, semS0, semS1):
    cid = lax.axis_index("c")
    sid = lax.axis_index("s")
    wid = sid * 2 + cid
    base0 = wid * EPW
    lane = lax.broadcasted_iota(jnp.int32, (16,), 0)

    idxs = ((srcb0, dstb0, ikb0, iqb0, ivb0, irb0),
            (srcb1, dstb1, ikb1, iqb1, ivb1, irb1))
    data = ((kb0, rkb0, qb0, vb0, rvb0),
            (kb1, rkb1, qb1, vb1, rvb1))
    semI = (semI0, semI1)
    semG = (semG0, semG1)
    semS = (semS0, semS1)
    ih = (src_h, dst_h, ik_h, iq_h, iv_h, ir_h)
    th = (kn_h, rtek_h, qr_h, vr_h, rtev_h)

    def fire_idx(c, p):
        for a in range(6):
            pltpu.async_copy(ih[a].at[pl.ds(base0 + c * CH, CH)], idxs[p][a], semI[p])

    def wait_idx(c, p):
        for a in range(6):
            pltpu.make_async_copy(ih[a].at[pl.ds(base0 + c * CH, CH)],
                                  idxs[p][a], semI[p]).wait()

    def gidx(p):
        return (idxs[p][0], idxs[p][2], idxs[p][3], idxs[p][4], idxs[p][5])

    def fire_gath(p):
        g = gidx(p)
        for a in range(5):
            pltpu.async_copy(th[a].at[g[a]], data[p][a], semG[p])

    def wait_gath(p):
        g = gidx(p)
        for a in range(5):
            pltpu.make_async_copy(th[a].at[g[a]], data[p][a], semG[p]).wait()

    def fire_store(c, p):
        pltpu.async_copy(data[p][0], m_h.at[pl.ds(base0 + c * CH, CH)], semS[p])

    def wait_store(c, p):
        pltpu.make_async_copy(data[p][0], m_h.at[pl.ds(base0 + c * CH, CH)], semS[p]).wait()

    hmask = lane < N_HEADS

    def compute(p):
        kb, rkb, qb, vb, rvb = data[p]
        dvec = idxs[p][1][pl.ds(0, 16)]
        for i in range(CH):
            s = []
            for h in range(N_HEADS):
                sl = pl.ds(h * D_K, 16)
                kq = (kb[i, sl] + rkb[i, sl]) * qb[i, sl]
                s.append(jnp.sum(kq))
            attv = jnp.zeros((16,), jnp.float32)
            for h in range(N_HEADS):
                attv = jnp.where(lane == h, s[h], attv)
            ev = jnp.exp(jnp.minimum(attv, 80.0))
            plsc.addupdate_scatter(denf, [dvec[i] * N_HEADS + lane], ev, mask=hmask)
            for h in range(N_HEADS):
                sl = pl.ds(h * D_K, 16)
                kb[i, sl] = (vb[i, sl] + rvb[i, sl]) * ev[h]

    def zden(i, carry):
        denf[pl.ds(i * 16, 16)] = jnp.zeros((16,), jnp.float32)
        return carry
    lax.fori_loop(0, DEN_W // 16, zden, 0)

    fire_idx(0, 0)
    wait_idx(0, 0)
    fire_gath(0)
    fire_idx(1, 1)

    wait_idx(1, 1)
    fire_gath(1)
    wait_gath(0)
    compute(0)
    fire_store(0, 0)
    fire_idx(2, 0)
    wait_idx(2, 0)
    wait_store(0, 0)
    fire_gath(0)
    wait_gath(1)
    compute(1)
    fire_store(1, 1)
    fire_idx(3, 1)

    def step(t, carry):
        g0 = 2 * t
        wait_store(g0 - 1, 1)
        wait_idx(g0 + 1, 1)
        fire_gath(1)
        wait_gath(0)
        compute(0)
        fire_store(g0, 0)
        fire_idx(g0 + 2, 0)
        wait_idx(g0 + 2, 0)
        wait_store(g0, 0)
        fire_gath(0)
        wait_gath(1)
        compute(1)
        fire_store(g0 + 1, 1)
        fire_idx(g0 + 3, 1)
        return carry

    lax.fori_loop(1, (NFULL - 1) // 2, step, 0)

    wait_store(NFULL - 2, 1)
    wait_gath(0)
    compute(0)
    fire_store(NFULL - 1, 0)
    wait_idx(NFULL, 1)
    wait_store(NFULL - 1, 0)

    pltpu.sync_copy(denf, denp_h.at[wid])


def _sc_edge(srcp, dstp, ikp, iqp, ivp, irp, kn, rtek, qr, vr, rtev):
    mesh = plsc.VectorSubcoreMesh(core_axis_name="c", subcore_axis_name="s")
    ib = [pltpu.VMEM((CH,), jnp.int32) for _ in range(12)]
    db = [pltpu.VMEM((CH, HID), jnp.float32) for _ in range(10)]
    f = functools.partial(
        pl.kernel,
        out_type=(jax.ShapeDtypeStruct((N_EDGES, HID), jnp.float32),
                  jax.ShapeDtypeStruct((NW, DEN_W), jnp.float32)),
        mesh=mesh,
        scratch_types=ib + db + [
            pltpu.VMEM((DEN_W,), jnp.float32),
            pltpu.SemaphoreType.DMA,
            pltpu.SemaphoreType.DMA,
            pltpu.SemaphoreType.DMA,
            pltpu.SemaphoreType.DMA,
            pltpu.SemaphoreType.DMA,
            pltpu.SemaphoreType.DMA,
        ],
        compiler_params=pltpu.CompilerParams(needs_layout_passes=False),
    )(_sc_edge_body)
    return f(srcp, dstp, ikp, iqp, ivp, irp, kn, rtek, qr, vr, rtev)


def _sc_scatter_body(m_h, dst_h, num_h, shared, tmp,
                     mab0, mab1, dstb0, dstb1, idxb0, idxb1, semL0, semL1):
    cid = lax.axis_index("c")
    sid = lax.axis_index("s")
    off = cid * HALF
    base0 = sid * EPS
    mab = (mab0, mab1)
    dstb = (dstb0, dstb1)
    idxb = (idxb0, idxb1)
    semL = (semL0, semL1)

    def fire_loads(c, p):
        @pl.when(c < NCHS)
        def _():
            pltpu.async_copy(dst_h.at[pl.ds(base0 + c * SCH, SCH)], dstb[p], semL[p])
            pltpu.async_copy(m_h.at[pl.ds(base0 + c * SCH, SCH)], mab[p], semL[p])

    def wait_loads(c, p):
        pltpu.make_async_copy(dst_h.at[pl.ds(base0 + c * SCH, SCH)], dstb[p], semL[p]).wait()
        pltpu.make_async_copy(m_h.at[pl.ds(base0 + c * SCH, SCH)], mab[p], semL[p]).wait()

    def do_scatter(p):
        for j in range(SCH // 16):
            sl = pl.ds(j * 16, 16)
            local = dstb[p][sl] - off
            ok = (local >= 0) & (local < HALF)
            idxb[p][sl] = jnp.where(ok, local, HALF)
        pltpu.sync_copy(mab[p], shared.at[idxb[p]], add=True)

    def zrow(i, carry):
        for j in range(HID // 16):
            tmp[i, pl.ds(j * 16, 16)] = jnp.zeros((16,), jnp.float32)
        return carry

    lax.fori_loop(0, ZROWS, zrow, 0)
    pltpu.sync_copy(tmp.at[pl.ds(0, ZROWS)], shared.at[pl.ds(sid * ZROWS, ZROWS)])
    plsc.subcore_barrier()

    fire_loads(0, 0)
    fire_loads(1, 1)

    def step(t, carry):
        c0 = 2 * t
        wait_loads(c0, 0)
        fire_loads(c0 + 2, 0)
        do_scatter(0)
        wait_loads(c0 + 1, 1)
        fire_loads(c0 + 3, 1)
        do_scatter(1)
        return carry

    lax.fori_loop(0, NCHS // 2, step, 0)
    plsc.subcore_barrier()
    pltpu.sync_copy(shared.at[pl.ds(sid * DROWS, DROWS)], tmp.at[pl.ds(0, DROWS)])
    pltpu.sync_copy(tmp.at[pl.ds(0, DROWS)], num_h.at[pl.ds(off + sid * DROWS, DROWS)])


def _sc_scatter(m, dst):
    mesh = plsc.VectorSubcoreMesh(core_axis_name="c", subcore_axis_name="s")
    f = functools.partial(
        pl.kernel,
        out_type=jax.ShapeDtypeStruct((NPAD, HID), jnp.float32),
        mesh=mesh,
        scratch_types=[
            pltpu.VMEM_SHARED((ACC_ROWS, HID), jnp.float32),
            pltpu.VMEM((ZROWS, HID), jnp.float32),
            pltpu.VMEM((SCH, HID), jnp.float32),
            pltpu.VMEM((SCH, HID), jnp.float32),
            pltpu.VMEM((SCH,), jnp.int32),
            pltpu.VMEM((SCH,), jnp.int32),
            pltpu.VMEM((SCH,), jnp.int32),
            pltpu.VMEM((SCH,), jnp.int32),
            pltpu.SemaphoreType.DMA,
            pltpu.SemaphoreType.DMA,
        ],
        compiler_params=pltpu.CompilerParams(needs_layout_passes=False),
    )(_sc_scatter_body)
    return f(m, dst)


def kernel(node_feature, node_type, edge_time, edge_index, edge_type,
           adapt_W, adapt_b, Wk, bk, Wq, bq, Wv, bv, Wa, ba,
           rel_pri, rel_att, rel_msg, skip, rte_W, rte_b, cls_W, cls_b):
    nt = node_type.astype(jnp.int32)
    src = edge_index[0].astype(jnp.int32)
    dst = edge_index[1].astype(jnp.int32)
    et = edge_type.astype(jnp.int32)
    tm = edge_time.astype(jnp.int32)

    stype = nt[src]
    ik = stype * 128 + tm
    iq = dst * N_REL + et
    iv = src * N_REL + et
    ir = ik * N_REL + et

    def padE(a):
        return jnp.pad(a, (0, EPAD - N_EDGES))

    srcp, dstp, ikp, iqp, ivp, irp = (padE(a) for a in (src, dst, ik, iq, iv, ir))

    oh = (nt[:, None] == jnp.arange(N_TYPES, dtype=jnp.int32)[None, :]).astype(jnp.float32)
    oh_p = jnp.pad(oh, ((0, NPAD - N_NODES), (0, 0)))
    nf_p = jnp.pad(node_feature, ((0, NPAD - N_NODES), (0, 0)))

    pe = _pe_table()
    sig = jax.nn.sigmoid(skip)

    x = _adapt(nf_p, oh_p, adapt_W, adapt_b)

    for l in range(N_LAYERS):
        scale = jnp.repeat(rel_pri[l], D_K, axis=-1) / math.sqrt(D_K)
        Aatt = _blockdiag(jnp.swapaxes(rel_att[l], -1, -2)) * scale[:, None, :]
        Amsg = _blockdiag(rel_msg[l])

        rtek, rtev = _tables(pe, rte_W[l], rte_b[l][None, :], Wk[l], Wv[l], Amsg)
        kn, qr, vr = _nodepre(x, oh_p, Wk[l], bk[l], Wq[l], bq[l], Wv[l], bv[l], Aatt, Amsg)

        m, denp = _sc_edge(srcp, dstp, ikp, iqp, ivp, irp, kn, rtek, qr, vr, rtev)
        num = _sc_scatter(m, dst)
        denp2 = jnp.pad(denp.reshape(NW, N_NODES, N_HEADS),
                        ((0, 0), (0, NPAD - N_NODES), (0, 0)))
        x = _update(num, denp2, x, oh_p, Wa[l], ba[l], sig[l][:, None])

    out = _cls(x, cls_W, cls_b[None, :])
    return out[:N_NODES]
